# Initial kernel scaffold; baseline (speedup 1.0000x reference)
import jax
import jax.numpy as jnp
from jax.experimental import pallas as pl


def kernel(x, edge_index, edge_attr, Wm_b, bm_b, Wu_b, bu_b, Wm_f, bm_f, Wu_f, bu_f):
    raise NotImplementedError("write your pallas kernel here")


# speedup vs baseline: 3.7131x; 3.7131x over previous
import functools

import jax
import jax.numpy as jnp
from jax import lax
from jax.experimental import pallas as pl
from jax.experimental.pallas import tpu as pltpu
from jax.experimental.pallas import tpu_sc as plsc

N = 10000
E = 320000
D = 128
DE = 16

NC = 2
NS = 16
NW = NC * NS

G = 128
GROUPS = E // G
GPW = 80
E_PAD = GPW * NW * G
SUPER = 8
STRIDE = 640


def _mm_pre_node(x, Wmx, bm, Wux):
    BLK = 1000

    def body(x_ref, wm_ref, b_ref, wu_ref, y_ref, z_ref):
        xb = x_ref[...]
        y_ref[...] = jnp.dot(xb, wm_ref[...], preferred_element_type=jnp.float32) + b_ref[...]
        z_ref[...] = jnp.dot(xb, wu_ref[...], preferred_element_type=jnp.float32)

    return pl.pallas_call(
        body,
        grid=(N // BLK,),
        in_specs=[
            pl.BlockSpec((BLK, D), lambda i: (i, 0)),
            pl.BlockSpec((D, D), lambda i: (0, 0)),
            pl.BlockSpec((1, D), lambda i: (0, 0)),
            pl.BlockSpec((D, D), lambda i: (0, 0)),
        ],
        out_specs=[
            pl.BlockSpec((BLK, D), lambda i: (i, 0)),
            pl.BlockSpec((BLK, D), lambda i: (i, 0)),
        ],
        out_shape=[jax.ShapeDtypeStruct((N, D), jnp.float32)] * 2,
    )(x, Wmx, bm.reshape(1, D), Wux)


def _mm_edges(ea, We_b, We_f):
    BLK = 4000

    def body(ea_ref, wb_ref, wf_ref, eb_ref, ef_ref):
        a = ea_ref[...]
        eb_ref[...] = jnp.dot(a, wb_ref[...], preferred_element_type=jnp.float32)
        ef_ref[...] = jnp.dot(a, wf_ref[...], preferred_element_type=jnp.float32)

    return pl.pallas_call(
        body,
        grid=(E // BLK,),
        in_specs=[
            pl.BlockSpec((BLK, DE), lambda i: (i, 0)),
            pl.BlockSpec((DE, D), lambda i: (0, 0)),
            pl.BlockSpec((DE, D), lambda i: (0, 0)),
        ],
        out_specs=[
            pl.BlockSpec((BLK, D), lambda i: (i, 0)),
            pl.BlockSpec((BLK, D), lambda i: (i, 0)),
        ],
        out_shape=[jax.ShapeDtypeStruct((E, D), jnp.float32)] * 2,
    )(ea, We_b, We_f)


def _mm_update(z, agg0, agg1, Wua, bu, Wmx2, bm2, Wux2):
    BLK = 1000

    def body(z_ref, a0_ref, a1_ref, wua_ref, bu_ref, wm_ref, bm_ref, wu_ref,
             y2_ref, z2_ref):
        agg = a0_ref[...] + a1_ref[...]
        h = jnp.maximum(
            z_ref[...] + jnp.dot(agg, wua_ref[...], preferred_element_type=jnp.float32)
            + bu_ref[...], 0.0)
        y2_ref[...] = jnp.dot(h, wm_ref[...], preferred_element_type=jnp.float32) + bm_ref[...]
        z2_ref[...] = jnp.dot(h, wu_ref[...], preferred_element_type=jnp.float32)

    return pl.pallas_call(
        body,
        grid=(N // BLK,),
        in_specs=[
            pl.BlockSpec((BLK, D), lambda i: (i, 0)),
            pl.BlockSpec((BLK, D), lambda i: (i, 0)),
            pl.BlockSpec((BLK, D), lambda i: (i, 0)),
            pl.BlockSpec((D, D), lambda i: (0, 0)),
            pl.BlockSpec((1, D), lambda i: (0, 0)),
            pl.BlockSpec((D, D), lambda i: (0, 0)),
            pl.BlockSpec((1, D), lambda i: (0, 0)),
            pl.BlockSpec((D, D), lambda i: (0, 0)),
        ],
        out_specs=[
            pl.BlockSpec((BLK, D), lambda i: (i, 0)),
            pl.BlockSpec((BLK, D), lambda i: (i, 0)),
        ],
        out_shape=[jax.ShapeDtypeStruct((N, D), jnp.float32)] * 2,
    )(z, agg0, agg1, Wua, bu.reshape(1, D), Wmx2, bm2.reshape(1, D), Wux2)


def _mm_final(z, agg0, agg1, Wua, bu):
    BLK = 1000

    def body(z_ref, a0_ref, a1_ref, wua_ref, bu_ref, o_ref):
        agg = a0_ref[...] + a1_ref[...]
        o_ref[...] = jnp.maximum(
            z_ref[...] + jnp.dot(agg, wua_ref[...], preferred_element_type=jnp.float32)
            + bu_ref[...], 0.0)

    return pl.pallas_call(
        body,
        grid=(N // BLK,),
        in_specs=[
            pl.BlockSpec((BLK, D), lambda i: (i, 0)),
            pl.BlockSpec((BLK, D), lambda i: (i, 0)),
            pl.BlockSpec((BLK, D), lambda i: (i, 0)),
            pl.BlockSpec((D, D), lambda i: (0, 0)),
            pl.BlockSpec((1, D), lambda i: (0, 0)),
        ],
        out_specs=pl.BlockSpec((BLK, D), lambda i: (i, 0)),
        out_shape=jax.ShapeDtypeStruct((N, D), jnp.float32),
    )(z, agg0, agg1, Wua, bu.reshape(1, D))


def _sc_layer(y, src, dst, eb):
    mesh = plsc.VectorSubcoreMesh(core_axis_name="c", subcore_axis_name="s")

    @functools.partial(
        pl.kernel,
        mesh=mesh,
        out_type=jax.ShapeDtypeStruct((NC, N, D), jnp.float32),
        scratch_types=[
            pltpu.VMEM((SUPER, G), jnp.int32),
            pltpu.VMEM((SUPER, G), jnp.int32),
            pltpu.VMEM((G, D), jnp.float32),
            pltpu.VMEM((G, D), jnp.float32),
            pltpu.VMEM_SHARED((N, D), jnp.float32),
            pltpu.SemaphoreType.DMA,
            pltpu.SemaphoreType.DMA,
        ],
    )
    def k(y_hbm, src_hbm, dst_hbm, eb_hbm, out_hbm,
          src_v, dst_v, rows_v, e_v, agg_sh, sem, sem2):
        cid = lax.axis_index("c")
        sid = lax.axis_index("s")
        wid = cid * NS + sid
        g0 = wid * GPW

        def zero_body(i, _):
            for j in range(D // 16):
                e_v[i, pl.ds(j * 16, 16)] = jnp.zeros((16,), jnp.float32)
            return 0
        lax.fori_loop(0, G, zero_body, 0)

        row0 = sid * STRIDE

        @pl.when(sid < NS - 1)
        def _():
            for r in range(STRIDE // G):
                pltpu.sync_copy(e_v, agg_sh.at[pl.ds(row0 + r * G, G)])

        @pl.when(sid == NS - 1)
        def _():
            base = (NS - 1) * STRIDE
            left = N - base
            for r in range(left // G):
                pltpu.sync_copy(e_v, agg_sh.at[pl.ds(base + r * G, G)])
            rem = left % G
            if rem:
                pltpu.sync_copy(e_v.at[pl.ds(0, rem)],
                                agg_sh.at[pl.ds(base + left - rem, rem)])
        plsc.subcore_barrier()

        def super_body(sc, _):
            gbase = g0 + sc * SUPER
            pltpu.sync_copy(src_hbm.at[pl.ds(gbase, SUPER)], src_v)
            pltpu.sync_copy(dst_hbm.at[pl.ds(gbase, SUPER)], dst_v)

            def chunk_body(ci, _):
                g = gbase + ci

                @pl.when(g < GROUPS)
                def _():
                    cp_e = pltpu.async_copy(eb_hbm.at[pl.ds(g * G, G)], e_v, sem2)
                    cp_g = pltpu.async_copy(y_hbm.at[src_v.at[ci]], rows_v, sem)
                    cp_e.wait()
                    cp_g.wait()

                    def edge_body(i, _):
                        for j in range(D // 16):
                            sl = pl.ds(j * 16, 16)
                            rows_v[i, sl] = jnp.maximum(
                                rows_v[i, sl] + e_v[i, sl], 0.0)
                        return 0
                    lax.fori_loop(0, G, edge_body, 0)

                    pltpu.sync_copy(rows_v, agg_sh.at[dst_v.at[ci]], add=True)
                return 0
            lax.fori_loop(0, SUPER, chunk_body, 0)
            return 0
        lax.fori_loop(0, GPW // SUPER, super_body, 0)
        plsc.subcore_barrier()

        @pl.when(sid < NS - 1)
        def _():
            pltpu.sync_copy(agg_sh.at[pl.ds(row0, STRIDE)],
                            out_hbm.at[cid, pl.ds(row0, STRIDE)])

        @pl.when(sid == NS - 1)
        def _():
            base = (NS - 1) * STRIDE
            pltpu.sync_copy(agg_sh.at[pl.ds(base, N - base)],
                            out_hbm.at[cid, pl.ds(base, N - base)])

    return k(y, src.reshape(E_PAD // G, G), dst.reshape(E_PAD // G, G), eb)


def kernel(x, edge_index, edge_attr, Wm_b, bm_b, Wu_b, bu_b, Wm_f, bm_f, Wu_f, bu_f):
    pad = E_PAD - E
    src = jnp.concatenate([edge_index[0], jnp.zeros((pad,), jnp.int32)])
    dst = jnp.concatenate([edge_index[1], jnp.zeros((pad,), jnp.int32)])

    e_b, e_f = _mm_edges(edge_attr, Wm_b[D:], Wm_f[D:])
    y_b, z_b = _mm_pre_node(x, Wm_b[:D], bm_b, Wu_b[:D])
    parts_b = _sc_layer(y_b, src, dst, e_b)
    y_f, z_f = _mm_update(z_b, parts_b[0], parts_b[1], Wu_b[D:], bu_b,
                          Wm_f[:D], bm_f, Wu_f[:D])
    parts_f = _sc_layer(y_f, src, dst, e_f)
    return _mm_final(z_f, parts_f[0], parts_f[1], Wu_f[D:], bu_f)
